# Initial kernel scaffold; baseline (speedup 1.0000x reference)
import jax
import jax.numpy as jnp
from jax.experimental import pallas as pl


def kernel(x, edge_index, W_proj, b_proj, W_msg, b_msg, w_ih, w_hh, b_ih, b_hh, W_out, b_out):
    raise NotImplementedError("write your pallas kernel here")


# speedup vs baseline: 7.1493x; 7.1493x over previous
import functools

import jax
import jax.numpy as jnp
from jax import lax
from jax.experimental import pallas as pl
from jax.experimental.pallas import tpu as pltpu
from jax.experimental.pallas import tpu_sc as plsc

N = 10000
E = 320000
D = 128
H = 128
OUT = 128
STEPS = 3

NC = 2
NS = 16
NW = NC * NS
EPW = E // NW
CHUNK = 80
NCH = EPW // CHUNK
N_PAD = 10240
ROWS_PER_TILE = N_PAD // NS

@functools.cache
def _get_sc_segment_sum():
    mesh = plsc.VectorSubcoreMesh(core_axis_name="c", subcore_axis_name="s",
                                  num_cores=NC, num_subcores=NS)

    @functools.partial(
        pl.kernel,
        out_type=jax.ShapeDtypeStruct((NC, N_PAD, H), jnp.float32),
        mesh=mesh,
        scratch_types=[
            pltpu.VMEM((NCH, CHUNK), jnp.int32),
            pltpu.VMEM((NCH, CHUNK), jnp.int32),
            pltpu.VMEM((CHUNK, H), jnp.float32),
            pltpu.VMEM_SHARED((N_PAD, H), jnp.float32),
            pltpu.SemaphoreType.DMA,
        ],
    )
    def _sc_segment_sum(m_hbm, src_hbm, dst_hbm, zero_hbm, out_hbm,
                        src_v, dst_v, rows_v, acc_sh, sem):
        cid = lax.axis_index("c")
        sid = lax.axis_index("s")
        wid = sid * NC + cid
        pltpu.sync_copy(src_hbm.at[wid], src_v)
        pltpu.sync_copy(dst_hbm.at[wid], dst_v)
        band = pl.ds(sid * ROWS_PER_TILE, ROWS_PER_TILE)
        pltpu.sync_copy(zero_hbm.at[band], acc_sh.at[band])
        plsc.subcore_barrier()

        def body(j, carry):
            pltpu.async_copy(m_hbm.at[src_v.at[j]], rows_v, sem).wait()
            pltpu.sync_copy(rows_v, acc_sh.at[dst_v.at[j]], add=True)
            return carry

        lax.fori_loop(0, NCH, body, 0)
        plsc.subcore_barrier()
        pltpu.sync_copy(acc_sh.at[band], out_hbm.at[cid, band])

    return _sc_segment_sum


R = 2000
GRID = N // R


def _init_body(x_ref, wpT, bp, wmT, bm, h_ref, m_ref):
    h = jnp.maximum(
        jnp.dot(x_ref[:], wpT[:], preferred_element_type=jnp.float32) + bp[:],
        0.0)
    h_ref[:] = h
    m_ref[:] = jnp.dot(h, wmT[:], preferred_element_type=jnp.float32) + bm[:]


_init_call = pl.pallas_call(
    _init_body,
    grid=(GRID,),
    in_specs=[
        pl.BlockSpec((R, D), lambda i: (i, 0)),
        pl.BlockSpec((D, H), lambda i: (0, 0)),
        pl.BlockSpec((1, H), lambda i: (0, 0)),
        pl.BlockSpec((H, H), lambda i: (0, 0)),
        pl.BlockSpec((1, H), lambda i: (0, 0)),
    ],
    out_specs=[
        pl.BlockSpec((R, H), lambda i: (i, 0)),
        pl.BlockSpec((R, H), lambda i: (i, 0)),
    ],
    out_shape=[
        jax.ShapeDtypeStruct((N, H), jnp.float32),
        jax.ShapeDtypeStruct((N, H), jnp.float32),
    ],
)


def _gru(parts, h, gi_w, gh_w, bih, bhh):
    a = parts[0] + parts[1]
    gi = jnp.dot(a, gi_w, preferred_element_type=jnp.float32) + bih
    gh = jnp.dot(h, gh_w, preferred_element_type=jnp.float32) + bhh
    r = jax.nn.sigmoid(gi[:, :H] + gh[:, :H])
    z = jax.nn.sigmoid(gi[:, H:2 * H] + gh[:, H:2 * H])
    n = jnp.tanh(gi[:, 2 * H:] + r * gh[:, 2 * H:])
    return (1.0 - z) * n + z * h


def _step_body(parts_ref, h_ref, wihT, whhT, bih, bhh, wmT, bm, hout, mout):
    hn = _gru(parts_ref[:], h_ref[:], wihT[:], whhT[:], bih[:], bhh[:])
    hout[:] = hn
    mout[:] = jnp.dot(hn, wmT[:], preferred_element_type=jnp.float32) + bm[:]


def _last_body(parts_ref, h_ref, wihT, whhT, bih, bhh, woT, bo, out_ref):
    hn = _gru(parts_ref[:], h_ref[:], wihT[:], whhT[:], bih[:], bhh[:])
    out_ref[:] = jnp.dot(hn, woT[:], preferred_element_type=jnp.float32) + bo[:]


_common_in_specs = [
    pl.BlockSpec((NC, R, H), lambda i: (0, i, 0)),
    pl.BlockSpec((R, H), lambda i: (i, 0)),
    pl.BlockSpec((H, 3 * H), lambda i: (0, 0)),
    pl.BlockSpec((H, 3 * H), lambda i: (0, 0)),
    pl.BlockSpec((1, 3 * H), lambda i: (0, 0)),
    pl.BlockSpec((1, 3 * H), lambda i: (0, 0)),
    pl.BlockSpec((H, H), lambda i: (0, 0)),
    pl.BlockSpec((1, H), lambda i: (0, 0)),
]

_step_call = pl.pallas_call(
    _step_body,
    grid=(GRID,),
    in_specs=_common_in_specs,
    out_specs=[
        pl.BlockSpec((R, H), lambda i: (i, 0)),
        pl.BlockSpec((R, H), lambda i: (i, 0)),
    ],
    out_shape=[
        jax.ShapeDtypeStruct((N, H), jnp.float32),
        jax.ShapeDtypeStruct((N, H), jnp.float32),
    ],
)

_last_call = pl.pallas_call(
    _last_body,
    grid=(GRID,),
    in_specs=_common_in_specs[:-2] + [
        pl.BlockSpec((H, OUT), lambda i: (0, 0)),
        pl.BlockSpec((1, OUT), lambda i: (0, 0)),
    ],
    out_specs=pl.BlockSpec((R, OUT), lambda i: (i, 0)),
    out_shape=jax.ShapeDtypeStruct((N, OUT), jnp.float32),
)


def kernel(x, edge_index, W_proj, b_proj, W_msg, b_msg, w_ih, w_hh,
           b_ih, b_hh, W_out, b_out):
    src = edge_index[0].astype(jnp.int32).reshape(NW, NCH, CHUNK)
    dst = edge_index[1].astype(jnp.int32).reshape(NW, NCH, CHUNK)
    zero = jnp.zeros((N_PAD, H), jnp.float32)
    wpT = W_proj.T
    wmT = W_msg.T
    wihT = w_ih.T
    whhT = w_hh.T
    woT = W_out.T
    bp = b_proj.reshape(1, H)
    bm = b_msg.reshape(1, H)
    bih = b_ih.reshape(1, 3 * H)
    bhh = b_hh.reshape(1, 3 * H)
    bo = b_out.reshape(1, OUT)

    sc_segment_sum = _get_sc_segment_sum()
    h, m = _init_call(x, wpT, bp, wmT, bm)
    out = None
    for step in range(STEPS):
        parts = sc_segment_sum(m, src, dst, zero)
        if step < STEPS - 1:
            h, m = _step_call(parts, h, wihT, whhT, bih, bhh, wmT, bm)
        else:
            out = _last_call(parts, h, wihT, whhT, bih, bhh, woT, bo)
    return out
